# Initial kernel scaffold; baseline (speedup 1.0000x reference)
import functools

import jax
import jax.numpy as jnp
from jax import lax
from jax.experimental import pallas as pl
from jax.experimental.pallas import tpu as pltpu
from jax.experimental.pallas import tpu_sc as plsc

NC = 2
NS = 16
NW = NC * NS

CHUNK = 128
D = 128

N_NODES = 10000
N_ACC = 10016
E_EDGES = 320000
CPT = 79
EPT = CPT * CHUNK
E_PAD = NW * EPT
ROWS_PER_TILE = N_NODES // NS
ZROWS_PER_TILE = N_ACC // NS

_MESH = plsc.VectorSubcoreMesh(
    core_axis_name="c", subcore_axis_name="s", num_cores=NC, num_subcores=NS)


def _seg_body(row_hbm, col_hbm, h_hbm, zeros_hbm, out_hbm,
              idx_r, idx_c, rows_v, acc, sem):
  cid = lax.axis_index("c")
  sid = lax.axis_index("s")
  wid = cid * NS + sid

  pltpu.sync_copy(zeros_hbm.at[pl.ds(sid * ZROWS_PER_TILE, ZROWS_PER_TILE)],
                  acc.at[pl.ds(sid * ZROWS_PER_TILE, ZROWS_PER_TILE)])
  plsc.subcore_barrier()

  def chunk_step(c, carry):
    pltpu.sync_copy(row_hbm.at[wid, c], idx_r)
    pltpu.sync_copy(col_hbm.at[wid, c], idx_c)
    pltpu.async_copy(h_hbm.at[idx_r], rows_v, sem).wait()
    pltpu.sync_copy(rows_v, acc.at[idx_c], add=True)
    return carry

  lax.fori_loop(0, CPT, chunk_step, 0)
  plsc.subcore_barrier()

  pltpu.sync_copy(acc.at[pl.ds(sid * ROWS_PER_TILE, ROWS_PER_TILE)],
                  out_hbm.at[cid, pl.ds(sid * ROWS_PER_TILE, ROWS_PER_TILE)])


_seg_sum = functools.partial(
    pl.kernel, _seg_body, mesh=_MESH,
    out_type=jax.ShapeDtypeStruct((NC, N_NODES, D), jnp.float32),
    scratch_types=[
        pltpu.VMEM((CHUNK,), jnp.int32),
        pltpu.VMEM((CHUNK,), jnp.int32),
        pltpu.VMEM((CHUNK, D), jnp.float32),
        pltpu.VMEM_SHARED((N_ACC, D), jnp.float32),
        pltpu.SemaphoreType.DMA,
    ],
)()


def _deg_body(col_hbm, zeros_hbm, out_hbm, idx_c, ones_v, acc):
  cid = lax.axis_index("c")
  sid = lax.axis_index("s")
  wid = cid * NS + sid

  for i in range(CHUNK // 16):
    ones_v[pl.ds(i * 16, 16)] = jnp.full((16,), 1.0, jnp.float32)

  pltpu.sync_copy(zeros_hbm.at[pl.ds(sid * ZROWS_PER_TILE, ZROWS_PER_TILE)],
                  acc.at[pl.ds(sid * ZROWS_PER_TILE, ZROWS_PER_TILE)])
  plsc.subcore_barrier()

  def chunk_step(c, carry):
    pltpu.sync_copy(col_hbm.at[wid, c], idx_c)
    pltpu.sync_copy(ones_v, acc.at[idx_c], add=True)
    return carry

  lax.fori_loop(0, CPT, chunk_step, 0)
  plsc.subcore_barrier()

  pltpu.sync_copy(acc.at[pl.ds(sid * ROWS_PER_TILE, ROWS_PER_TILE)],
                  out_hbm.at[cid, pl.ds(sid * ROWS_PER_TILE, ROWS_PER_TILE)])


_deg_sum = functools.partial(
    pl.kernel, _deg_body, mesh=_MESH,
    out_type=jax.ShapeDtypeStruct((NC, N_NODES), jnp.float32),
    scratch_types=[
        pltpu.VMEM((CHUNK,), jnp.int32),
        pltpu.VMEM((CHUNK,), jnp.float32),
        pltpu.VMEM_SHARED((N_ACC,), jnp.float32),
    ],
)()


ROW_BLK = 1000
GRID = N_NODES // ROW_BLK


def _stage_a_body(x_ref, w_ref, d0_ref, d1_ref, hp_ref, dis_ref):
  dis = lax.rsqrt(d0_ref[...] + d1_ref[...] + 1.0)
  h = jnp.dot(x_ref[...], w_ref[...], preferred_element_type=jnp.float32)
  hp_ref[...] = dis * h
  dis_ref[...] = dis


def _stage_a(x, w1, d0, d1):
  return pl.pallas_call(
      _stage_a_body,
      grid=(GRID,),
      in_specs=[
          pl.BlockSpec((ROW_BLK, D), lambda i: (i, 0)),
          pl.BlockSpec((D, D), lambda i: (0, 0)),
          pl.BlockSpec((ROW_BLK, 1), lambda i: (i, 0)),
          pl.BlockSpec((ROW_BLK, 1), lambda i: (i, 0)),
      ],
      out_specs=[
          pl.BlockSpec((ROW_BLK, D), lambda i: (i, 0)),
          pl.BlockSpec((ROW_BLK, 1), lambda i: (i, 0)),
      ],
      out_shape=[
          jax.ShapeDtypeStruct((N_NODES, D), jnp.float32),
          jax.ShapeDtypeStruct((N_NODES, 1), jnp.float32),
      ],
  )(x, w1, d0, d1)


def _stage_b_body(p0_ref, p1_ref, hp_ref, dis_ref, b_ref, w_ref, out_ref):
  dis = dis_ref[...]
  a = dis * (p0_ref[...] + p1_ref[...] + hp_ref[...]) + b_ref[...]
  a = jnp.maximum(a, 0.0)
  out_ref[...] = dis * jnp.dot(a, w_ref[...],
                               preferred_element_type=jnp.float32)


def _stage_b(p0, p1, hp, dis, b1, w2):
  return pl.pallas_call(
      _stage_b_body,
      grid=(GRID,),
      in_specs=[
          pl.BlockSpec((ROW_BLK, D), lambda i: (i, 0)),
          pl.BlockSpec((ROW_BLK, D), lambda i: (i, 0)),
          pl.BlockSpec((ROW_BLK, D), lambda i: (i, 0)),
          pl.BlockSpec((ROW_BLK, 1), lambda i: (i, 0)),
          pl.BlockSpec((1, D), lambda i: (0, 0)),
          pl.BlockSpec((D, D), lambda i: (0, 0)),
      ],
      out_specs=pl.BlockSpec((ROW_BLK, D), lambda i: (i, 0)),
      out_shape=jax.ShapeDtypeStruct((N_NODES, D), jnp.float32),
  )(p0, p1, hp, dis, b1, w2)


def _stage_c_body(p0_ref, p1_ref, hp_ref, dis_ref, b_ref, wfc_ref, bfc_ref,
                  out_ref, acc_ref):
  i = pl.program_id(0)

  @pl.when(i == 0)
  def _():
    acc_ref[...] = jnp.zeros_like(acc_ref)

  a = dis_ref[...] * (p0_ref[...] + p1_ref[...] + hp_ref[...]) + b_ref[...]
  acc_ref[...] += jnp.sum(a, axis=0, keepdims=True)

  @pl.when(i == GRID - 1)
  def _():
    g = acc_ref[...] * (1.0 / N_NODES)
    out_ref[...] = lax.dot_general(
        g, wfc_ref[...], (((1,), (1,)), ((), ())),
        preferred_element_type=jnp.float32) + bfc_ref[...]


def _stage_c(p0, p1, hp, dis, b2, wfc, bfc):
  return pl.pallas_call(
      _stage_c_body,
      grid=(GRID,),
      in_specs=[
          pl.BlockSpec((ROW_BLK, D), lambda i: (i, 0)),
          pl.BlockSpec((ROW_BLK, D), lambda i: (i, 0)),
          pl.BlockSpec((ROW_BLK, D), lambda i: (i, 0)),
          pl.BlockSpec((ROW_BLK, 1), lambda i: (i, 0)),
          pl.BlockSpec((1, D), lambda i: (0, 0)),
          pl.BlockSpec((40, D), lambda i: (0, 0)),
          pl.BlockSpec((1, 40), lambda i: (0, 0)),
      ],
      out_specs=pl.BlockSpec((1, 40), lambda i: (0, 0)),
      out_shape=jax.ShapeDtypeStruct((1, 40), jnp.float32),
      scratch_shapes=[pltpu.VMEM((1, D), jnp.float32)],
  )(p0, p1, hp, dis, b2, wfc, bfc)


def kernel(x, edge_index, W1, b1, W2, b2, Wfc, bfc):
  row = edge_index[0]
  col = edge_index[1]
  pad = E_PAD - E_EDGES
  row_p = jnp.concatenate([row, jnp.zeros((pad,), jnp.int32)])
  col_p = jnp.concatenate([col, jnp.full((pad,), N_NODES, jnp.int32)])
  row3 = row_p.reshape(NW, CPT, CHUNK)
  col3 = col_p.reshape(NW, CPT, CHUNK)

  zeros2 = jnp.zeros((N_ACC, D), jnp.float32)
  zeros1 = jnp.zeros((N_ACC,), jnp.float32)

  degp = _deg_sum(col3, zeros1)
  d0 = degp[0].reshape(N_NODES, 1)
  d1 = degp[1].reshape(N_NODES, 1)

  hp1, dis = _stage_a(x, W1, d0, d1)
  s1 = _seg_sum(row3, col3, hp1, zeros2)
  hp2 = _stage_b(s1[0], s1[1], hp1, dis,
                 b1.reshape(1, D), W2)
  s2 = _seg_sum(row3, col3, hp2, zeros2)
  out = _stage_c(s2[0], s2[1], hp2, dis,
                 b2.reshape(1, D), Wfc, bfc.reshape(1, 40))
  return out


# speedup vs baseline: 7.3803x; 7.3803x over previous
import functools

import jax
import jax.numpy as jnp
from jax import lax
from jax.experimental import pallas as pl
from jax.experimental.pallas import tpu as pltpu
from jax.experimental.pallas import tpu_sc as plsc

NC = 2
NS = 16
NW = NC * NS

CHUNK = 128
D = 128

N_NODES = 10000
HALF = 5000
ACC_ROWS = 5120
SLT = ACC_ROWS // NS
DUMMY = HALF

E_EDGES = 320000
EPT_RAW = E_EDGES // NS
CPT = (EPT_RAW + CHUNK - 1) // CHUNK
EPT = CPT * CHUNK
E_PAD = NS * EPT


def _seg_body(row_hbm, col_hbm, h_hbm, out_hbm,
              idx_r, idx_c, rows_v, stage_v, acc, sem):
  cid = lax.axis_index("c")
  sid = lax.axis_index("s")
  start = cid * HALF

  def zfill(i, carry):
    def zlane(j, c2):
      stage_v[i, pl.ds(j * 16, 16)] = jnp.zeros((16,), jnp.float32)
      return c2
    return lax.fori_loop(0, D // 16, zlane, carry)
  lax.fori_loop(0, SLT, zfill, 0)
  pltpu.sync_copy(stage_v, acc.at[pl.ds(sid * SLT, SLT)])
  plsc.subcore_barrier()

  def chunk_step(c, carry):
    pltpu.sync_copy(row_hbm.at[sid, c], idx_r)
    pltpu.sync_copy(col_hbm.at[sid, c], idx_c)
    def remap(j, c2):
      v = idx_c[pl.ds(j * 16, 16)] - start
      oob = (v < 0) | (v >= HALF)
      idx_c[pl.ds(j * 16, 16)] = jnp.where(oob, DUMMY, v)
      return c2
    lax.fori_loop(0, CHUNK // 16, remap, 0)
    pltpu.async_copy(h_hbm.at[idx_r], rows_v, sem).wait()
    pltpu.sync_copy(rows_v, acc.at[idx_c], add=True)
    return carry

  lax.fori_loop(0, CPT, chunk_step, 0)
  plsc.subcore_barrier()

  pltpu.sync_copy(acc.at[pl.ds(sid * SLT, SLT)], stage_v)
  pltpu.sync_copy(stage_v, out_hbm.at[pl.ds(cid * ACC_ROWS + sid * SLT, SLT)])


@functools.lru_cache(maxsize=None)
def _seg_sum_kernel():
  mesh = plsc.VectorSubcoreMesh(
      core_axis_name="c", subcore_axis_name="s",
      num_cores=NC, num_subcores=NS)
  return pl.kernel(
      _seg_body, mesh=mesh,
      out_type=jax.ShapeDtypeStruct((NC * ACC_ROWS, D), jnp.float32),
      scratch_types=[
          pltpu.VMEM((CHUNK,), jnp.int32),
          pltpu.VMEM((CHUNK,), jnp.int32),
          pltpu.VMEM((CHUNK, D), jnp.float32),
          pltpu.VMEM((SLT, D), jnp.float32),
          pltpu.VMEM_SHARED((ACC_ROWS, D), jnp.float32),
          pltpu.SemaphoreType.DMA,
      ],
  )


def _deg_body(col_hbm, out_hbm, idx_c, ones_v, stage_v, acc):
  cid = lax.axis_index("c")
  sid = lax.axis_index("s")
  start = cid * HALF

  for i in range(CHUNK // 16):
    ones_v[pl.ds(i * 16, 16)] = jnp.full((16,), 1.0, jnp.float32)

  def zfill(i, carry):
    stage_v[pl.ds(i * 16, 16)] = jnp.zeros((16,), jnp.float32)
    return carry
  lax.fori_loop(0, SLT // 16, zfill, 0)
  pltpu.sync_copy(stage_v, acc.at[pl.ds(sid * SLT, SLT)])
  plsc.subcore_barrier()

  def chunk_step(c, carry):
    pltpu.sync_copy(col_hbm.at[sid, c], idx_c)
    def remap(j, c2):
      v = idx_c[pl.ds(j * 16, 16)] - start
      oob = (v < 0) | (v >= HALF)
      idx_c[pl.ds(j * 16, 16)] = jnp.where(oob, DUMMY, v)
      return c2
    lax.fori_loop(0, CHUNK // 16, remap, 0)
    pltpu.sync_copy(ones_v, acc.at[idx_c], add=True)
    return carry

  lax.fori_loop(0, CPT, chunk_step, 0)
  plsc.subcore_barrier()

  pltpu.sync_copy(acc.at[pl.ds(sid * SLT, SLT)], stage_v)
  pltpu.sync_copy(stage_v, out_hbm.at[pl.ds(cid * ACC_ROWS + sid * SLT, SLT)])


@functools.lru_cache(maxsize=None)
def _deg_sum_kernel():
  mesh = plsc.VectorSubcoreMesh(
      core_axis_name="c", subcore_axis_name="s",
      num_cores=NC, num_subcores=NS)
  return pl.kernel(
      _deg_body, mesh=mesh,
      out_type=jax.ShapeDtypeStruct((NC * ACC_ROWS,), jnp.float32),
      scratch_types=[
          pltpu.VMEM((CHUNK,), jnp.int32),
          pltpu.VMEM((CHUNK,), jnp.float32),
          pltpu.VMEM((SLT,), jnp.float32),
          pltpu.VMEM_SHARED((ACC_ROWS,), jnp.float32),
      ],
  )


ROW_BLK = 1000
GRID = N_NODES // ROW_BLK


def _stage_a_body(x_ref, w_ref, d_ref, hp_ref, dis_ref):
  dis = lax.rsqrt(d_ref[...] + 1.0)
  h = jnp.dot(x_ref[...], w_ref[...], preferred_element_type=jnp.float32)
  hp_ref[...] = dis * h
  dis_ref[...] = dis


def _stage_a(x, w1, d):
  return pl.pallas_call(
      _stage_a_body,
      grid=(GRID,),
      in_specs=[
          pl.BlockSpec((ROW_BLK, D), lambda i: (i, 0)),
          pl.BlockSpec((D, D), lambda i: (0, 0)),
          pl.BlockSpec((ROW_BLK, 1), lambda i: (i, 0)),
      ],
      out_specs=[
          pl.BlockSpec((ROW_BLK, D), lambda i: (i, 0)),
          pl.BlockSpec((ROW_BLK, 1), lambda i: (i, 0)),
      ],
      out_shape=[
          jax.ShapeDtypeStruct((N_NODES, D), jnp.float32),
          jax.ShapeDtypeStruct((N_NODES, 1), jnp.float32),
      ],
  )(x, w1, d)


def _stage_b_body(p_ref, hp_ref, dis_ref, b_ref, w_ref, out_ref):
  dis = dis_ref[...]
  a = dis * (p_ref[...] + hp_ref[...]) + b_ref[...]
  a = jnp.maximum(a, 0.0)
  out_ref[...] = dis * jnp.dot(a, w_ref[...],
                               preferred_element_type=jnp.float32)


def _stage_b(p, hp, dis, b1, w2):
  return pl.pallas_call(
      _stage_b_body,
      grid=(GRID,),
      in_specs=[
          pl.BlockSpec((ROW_BLK, D), lambda i: (i, 0)),
          pl.BlockSpec((ROW_BLK, D), lambda i: (i, 0)),
          pl.BlockSpec((ROW_BLK, 1), lambda i: (i, 0)),
          pl.BlockSpec((1, D), lambda i: (0, 0)),
          pl.BlockSpec((D, D), lambda i: (0, 0)),
      ],
      out_specs=pl.BlockSpec((ROW_BLK, D), lambda i: (i, 0)),
      out_shape=jax.ShapeDtypeStruct((N_NODES, D), jnp.float32),
  )(p, hp, dis, b1, w2)


def _stage_c_body(p_ref, hp_ref, dis_ref, b_ref, wfc_ref, bfc_ref,
                  out_ref, acc_ref):
  i = pl.program_id(0)

  @pl.when(i == 0)
  def _():
    acc_ref[...] = jnp.zeros_like(acc_ref)

  a = dis_ref[...] * (p_ref[...] + hp_ref[...]) + b_ref[...]
  acc_ref[...] += jnp.sum(a, axis=0, keepdims=True)

  @pl.when(i == GRID - 1)
  def _():
    g = acc_ref[...] * (1.0 / N_NODES)
    out_ref[...] = lax.dot_general(
        g, wfc_ref[...], (((1,), (1,)), ((), ())),
        preferred_element_type=jnp.float32) + bfc_ref[...]


def _stage_c(p, hp, dis, b2, wfc, bfc):
  return pl.pallas_call(
      _stage_c_body,
      grid=(GRID,),
      in_specs=[
          pl.BlockSpec((ROW_BLK, D), lambda i: (i, 0)),
          pl.BlockSpec((ROW_BLK, D), lambda i: (i, 0)),
          pl.BlockSpec((ROW_BLK, 1), lambda i: (i, 0)),
          pl.BlockSpec((1, D), lambda i: (0, 0)),
          pl.BlockSpec((40, D), lambda i: (0, 0)),
          pl.BlockSpec((1, 40), lambda i: (0, 0)),
      ],
      out_specs=pl.BlockSpec((1, 40), lambda i: (0, 0)),
      out_shape=jax.ShapeDtypeStruct((1, 40), jnp.float32),
      scratch_shapes=[pltpu.VMEM((1, D), jnp.float32)],
  )(p, hp, dis, b2, wfc, bfc)


def _assemble(s):
  return jnp.concatenate([s[:HALF], s[ACC_ROWS:ACC_ROWS + HALF]], axis=0)


def kernel(x, edge_index, W1, b1, W2, b2, Wfc, bfc):
  row = edge_index[0]
  col = edge_index[1]
  pad = E_PAD - E_EDGES
  row_p = jnp.concatenate([row, jnp.zeros((pad,), jnp.int32)])
  col_p = jnp.concatenate([col, jnp.full((pad,), N_NODES, jnp.int32)])
  row3 = row_p.reshape(NS, CPT, CHUNK)
  col3 = col_p.reshape(NS, CPT, CHUNK)

  degp = _deg_sum_kernel()(col3)
  d = _assemble(degp.reshape(-1, 1))

  hp1, dis = _stage_a(x, W1, d)
  s1 = _assemble(_seg_sum_kernel()(row3, col3, hp1))
  hp2 = _stage_b(s1, hp1, dis, b1.reshape(1, D), W2)
  s2 = _assemble(_seg_sum_kernel()(row3, col3, hp2))
  out = _stage_c(s2, hp2, dis, b2.reshape(1, D), Wfc, bfc.reshape(1, 40))
  return out
